# baseline (speedup 1.0000x reference)
import jax
import jax.numpy as jnp
from jax.experimental import pallas as pl
from jax.experimental.pallas import tpu as pltpu

ROWS = 128
COLS = 32768
DROP = 512
KEEP = COLS - DROP
RB = 8
NCHUNK = ROWS // RB


def _tc_body(tok_hbm, out_ref, bufs, sems):
    copies = []
    for c in range(NCHUNK):
        cp = pltpu.make_async_copy(
            tok_hbm.at[pl.ds(c * RB, RB), :], bufs.at[c], sems.at[c])
        cp.start()
        copies.append(cp)
    for c in range(NCHUNK):
        copies[c].wait()
        x = bufs[c]
        full = jnp.sum(x, axis=1, keepdims=True)
        head = jnp.sum(x[:, :DROP], axis=1, keepdims=True)
        out_ref[pl.ds(c * RB, RB), :] = (full - head) * (1.0 / KEEP)


_tc_mean = pl.pallas_call(
    _tc_body,
    in_specs=[pl.BlockSpec(memory_space=pl.ANY)],
    out_specs=pl.BlockSpec(memory_space=pltpu.MemorySpace.VMEM),
    out_shape=jax.ShapeDtypeStruct((ROWS, 1), jnp.float32),
    scratch_shapes=[
        pltpu.VMEM((NCHUNK, RB, COLS), jnp.float32),
        pltpu.SemaphoreType.DMA((NCHUNK,)),
    ],
)


def kernel(tokens):
    return _tc_mean(tokens)


# speedup vs baseline: 1.1062x; 1.1062x over previous
import jax
import jax.numpy as jnp
from jax import lax
from jax.experimental import pallas as pl
from jax.experimental.pallas import tpu as pltpu

ROWS = 128
COLS = 32768
DROP = 512
KEEP = COLS - DROP
RB = 16
NCHUNK = ROWS // RB


def _tc_body(tok_hbm, out_ref, bufs, sems):
    copies = []
    for c in range(NCHUNK):
        cp = pltpu.make_async_copy(
            tok_hbm.at[pl.ds(c * RB, RB), :], bufs.at[c], sems.at[c])
        cp.start()
        copies.append(cp)
    cols = lax.broadcasted_iota(jnp.int32, (RB, COLS), 1)
    m = cols >= DROP
    for c in range(NCHUNK):
        copies[c].wait()
        x = jnp.where(m, bufs[c], 0.0)
        out_ref[pl.ds(c * RB, RB), :] = (
            jnp.sum(x, axis=1, keepdims=True) * (1.0 / KEEP))


_tc_mean = pl.pallas_call(
    _tc_body,
    in_specs=[pl.BlockSpec(memory_space=pl.ANY)],
    out_specs=pl.BlockSpec(memory_space=pltpu.MemorySpace.VMEM),
    out_shape=jax.ShapeDtypeStruct((ROWS, 1), jnp.float32),
    scratch_shapes=[
        pltpu.VMEM((NCHUNK, RB, COLS), jnp.float32),
        pltpu.SemaphoreType.DMA((NCHUNK,)),
    ],
)


def kernel(tokens):
    return _tc_mean(tokens)
